# baseline (speedup 1.0000x reference)
import functools

import jax
import jax.numpy as jnp
from jax import lax
from jax.experimental import pallas as pl
from jax.experimental.pallas import tpu as pltpu
from jax.experimental.pallas import tpu_sc as plsc

_N, _C, _H, _W = 4, 256, 14, 14
_P = _H * _W
_B = _N * _P
_S = 512
_NELEM = _N * _C * _P

_DIMS = (((1,), (0,)), ((), ()))

_BPAD = 1024
_NWORK = 32
_BPW = _BPAD // _NWORK


def _tc_body(x_ref, cb_ref, loss_ref, idx_ref):
    cb = cb_ref[...]
    c2 = jnp.sum(cb * cb, axis=1, keepdims=True)
    iota = jax.lax.broadcasted_iota(jnp.int32, (_S, _P), 0)
    acc = jnp.float32(0.0)
    for n in range(_N):
        xn = x_ref[n]
        dot = jax.lax.dot_general(cb, xn, _DIMS,
                                  preferred_element_type=jnp.float32,
                                  precision=jax.lax.Precision.HIGHEST)
        scores = c2 - 2.0 * dot
        minval = jnp.min(scores, axis=0)
        idx = jnp.min(jnp.where(scores == minval[None, :], iota, _S),
                      axis=0)
        idx_ref[n, 0, :] = idx
        x2 = jnp.sum(xn * xn, axis=0)
        acc += jnp.sum(x2 + minval)
    loss_ref[...] = jnp.reshape(acc / _NELEM, (1, 1))


def _sc_gather(cb_hbm, idx_hbm, out_hbm, idx_v, rows_v, sem):
    wid = lax.axis_index("s") * 2 + lax.axis_index("c")
    base = wid * _BPW
    pltpu.sync_copy(idx_hbm.at[pl.ds(base, _BPW)], idx_v)
    pltpu.async_copy(cb_hbm.at[idx_v], rows_v, sem).wait()
    pltpu.sync_copy(rows_v, out_hbm.at[pl.ds(base, _BPW)])


_sc_gather_call = functools.partial(
    pl.kernel,
    mesh=plsc.VectorSubcoreMesh(core_axis_name="c", subcore_axis_name="s"),
    out_type=jax.ShapeDtypeStruct((_BPAD, _C), jnp.float32),
    scratch_types=[
        pltpu.VMEM((_BPW,), jnp.int32),
        pltpu.VMEM((_BPW, _C), jnp.float32),
        pltpu.SemaphoreType.DMA,
    ],
)(_sc_gather)


@functools.partial(jax.jit, static_argnames=())
def kernel(x, codebook):
    x_flat = x.reshape(_N, _C, _P)
    loss2d, idx3d = pl.pallas_call(
        _tc_body,
        out_shape=(
            jax.ShapeDtypeStruct((1, 1), jnp.float32),
            jax.ShapeDtypeStruct((_N, 1, _P), jnp.int32),
        ),
    )(x_flat, codebook)
    idx_flat = jnp.pad(idx3d.reshape(_B), (0, _BPAD - _B))
    rows = _sc_gather_call(codebook, idx_flat)
    loss = loss2d[0, 0]
    codebook_indices = idx3d.reshape(_N, _H, _W)
    output = jnp.transpose(rows[:_B].reshape(_N, _P, _C), (0, 2, 1)).reshape(
        _N, _C, _H, _W)
    return (loss, codebook_indices, output)


# speedup vs baseline: 1.3225x; 1.3225x over previous
import functools

import jax
import jax.numpy as jnp
from jax import lax
from jax.experimental import pallas as pl
from jax.experimental.pallas import tpu as pltpu
from jax.experimental.pallas import tpu_sc as plsc

_N, _C, _H, _W = 4, 256, 14, 14
_P = _H * _W
_B = _N * _P
_S = 512
_NELEM = _N * _C * _P

_DIMS = (((1,), (0,)), ((), ()))

_NWORK = 14
_BPW = _B // _NWORK


def _tc_body(x_ref, cb_ref, loss_ref, idx_ref):
    cb = cb_ref[...]
    c2 = jnp.sum(cb * cb, axis=1, keepdims=True)
    iota = jax.lax.broadcasted_iota(jnp.int32, (_S, _P), 0)
    acc = jnp.float32(0.0)
    for n in range(_N):
        xn = x_ref[n]
        dot = jax.lax.dot_general(cb, xn, _DIMS,
                                  preferred_element_type=jnp.float32,
                                  precision=jax.lax.Precision.HIGHEST)
        scores = c2 - 2.0 * dot
        minval = jnp.min(scores, axis=0)
        idx = jnp.min(jnp.where(scores == minval[None, :], iota, _S),
                      axis=0)
        idx_ref[n, 0, :] = idx
        x2 = jnp.sum(xn * xn, axis=0)
        acc += jnp.sum(x2 + minval)
    loss_ref[...] = jnp.reshape(acc / _NELEM, (1, 1))


def _sc_gather(cb_hbm, idx_hbm, out_hbm, idx_v, rows_v, sem):
    wid = lax.axis_index("s") * 2 + lax.axis_index("c")

    @pl.when(wid < _NWORK)
    def _():
        base = wid * _BPW
        pltpu.sync_copy(idx_hbm.at[pl.ds(base, _BPW)], idx_v)
        pltpu.async_copy(cb_hbm.at[idx_v], rows_v, sem).wait()
        pltpu.sync_copy(rows_v, out_hbm.at[pl.ds(base, _BPW)])


_sc_gather_call = functools.partial(
    pl.kernel,
    mesh=plsc.VectorSubcoreMesh(core_axis_name="c", subcore_axis_name="s"),
    out_type=jax.ShapeDtypeStruct((_B, _C), jnp.float32),
    scratch_types=[
        pltpu.VMEM((_BPW,), jnp.int32),
        pltpu.VMEM((_BPW, _C), jnp.float32),
        pltpu.SemaphoreType.DMA,
    ],
)(_sc_gather)


@functools.partial(jax.jit, static_argnames=())
def kernel(x, codebook):
    x_flat = x.reshape(_N, _C, _P)
    loss2d, idx3d = pl.pallas_call(
        _tc_body,
        out_shape=(
            jax.ShapeDtypeStruct((1, 1), jnp.float32),
            jax.ShapeDtypeStruct((_N, 1, _P), jnp.int32),
        ),
    )(x_flat, codebook)
    idx_flat = idx3d.reshape(_B)
    rows = _sc_gather_call(codebook, idx_flat)
    loss = loss2d[0, 0]
    codebook_indices = idx3d.reshape(_N, _H, _W)
    output = jnp.transpose(rows.reshape(_N, _P, _C), (0, 2, 1)).reshape(
        _N, _C, _H, _W)
    return (loss, codebook_indices, output)
